# baseline (speedup 1.0000x reference)
import functools

import jax
import jax.numpy as jnp
from jax import lax
from jax.experimental import pallas as pl
from jax.experimental.pallas import tpu as pltpu
from jax.experimental.pallas import tpu_sc as plsc

_NC = 2
_NS = 16
_NW = _NC * _NS

_N_NODES = 10000
_D = 128
_CH = 128
_NCH = 80
_E_PAD = _NW * _NCH * _CH
_ACC_N = 10240
_STRIPE = _ACC_N // _NS
_DUMMY = _N_NODES


def _sc_body(feat, pk, out, pkv, srcb, dstb, rows, acc, gs0, gs1, ss0, ss1):
    c = lax.axis_index("c")
    s = lax.axis_index("s")
    slab = c * _NS + s
    gsems = (gs0, gs1)
    ssems = (ss0, ss1)

    zeros16 = jnp.zeros((16,), jnp.float32)

    def zrow(r, carry):
        for g in range(8):
            rows[0, r, pl.ds(g * 16, 16)] = zeros16
        return carry

    lax.fori_loop(0, _CH, zrow, 0)
    base = s * _STRIPE
    for i in range(_STRIPE // _CH):
        pltpu.sync_copy(rows.at[0], acc.at[pl.ds(base + i * _CH, _CH)])
    plsc.subcore_barrier()

    pltpu.sync_copy(pk.at[slab], pkv)

    def unpack(j, b):
        for g in range(8):
            v = pkv[j, pl.ds(g * 16, 16)]
            srcb[b, pl.ds(g * 16, 16)] = v & 0xFFFF
            dstb[b, pl.ds(g * 16, 16)] = v >> 16

    def gather_start(b):
        pltpu.async_copy(feat.at[srcb.at[b]], rows.at[b], gsems[b])

    def gather_wait(b):
        pltpu.make_async_copy(feat.at[srcb.at[b]], rows.at[b], gsems[b]).wait()

    def scatter_start(b):
        pltpu.async_copy(rows.at[b], acc.at[dstb.at[b]], ssems[b], add=True)

    def scatter_wait(b):
        pltpu.make_async_copy(rows.at[b], acc.at[dstb.at[b]], ssems[b]).wait()

    for b in range(2):
        unpack(b, b)
        gather_start(b)

    def body(i, carry):
        for b in range(2):
            j = i * 2 + b
            gather_wait(b)
            scatter_start(b)
            nxt = j + 2

            @pl.when(nxt < _NCH)
            def _():
                scatter_wait(b)
                unpack(nxt, b)
                gather_start(b)

        return carry

    lax.fori_loop(0, _NCH // 2, body, 0)
    for b in range(2):
        scatter_wait(b)

    plsc.subcore_barrier()
    pltpu.sync_copy(acc.at[pl.ds(base, _STRIPE)], out.at[c, pl.ds(base, _STRIPE)])


@jax.jit
def _sc_aggregate(feature, pk3):
    mesh = plsc.VectorSubcoreMesh(core_axis_name="c", subcore_axis_name="s")
    k = functools.partial(
        pl.kernel,
        mesh=mesh,
        out_type=jax.ShapeDtypeStruct((_NC, _ACC_N, _D), jnp.float32),
        scratch_types=[
            pltpu.VMEM((_NCH, _CH), jnp.int32),
            pltpu.VMEM((2, _CH), jnp.int32),
            pltpu.VMEM((2, _CH), jnp.int32),
            pltpu.VMEM((2, _CH, _D), jnp.float32),
            pltpu.VMEM_SHARED((_ACC_N, _D), jnp.float32),
            pltpu.SemaphoreType.DMA,
            pltpu.SemaphoreType.DMA,
            pltpu.SemaphoreType.DMA,
            pltpu.SemaphoreType.DMA,
        ],
    )(_sc_body)
    return k(feature, pk3)


def _pack_edges(ei3, n_edges):
    def body(e_ref, o_ref):
        e = e_ref[...]
        src = e[0]
        dst = e[1]
        pid = pl.program_id(0)
        rows_i = jax.lax.broadcasted_iota(jnp.int32, (_NCH, _CH), 0)
        cols_i = jax.lax.broadcasted_iota(jnp.int32, (_NCH, _CH), 1)
        gid = pid * (_NCH * _CH) + rows_i * _CH + cols_i
        valid = gid < n_edges
        src = jnp.where(valid, src, gid % _N_NODES)
        dst = jnp.where(valid, dst, _DUMMY + gid % (_ACC_N - _N_NODES))
        o_ref[0] = src | (dst << 16)

    return pl.pallas_call(
        body,
        grid=(_NW,),
        in_specs=[pl.BlockSpec((2, _NCH, _CH), lambda i: (0, i, 0))],
        out_specs=pl.BlockSpec((1, _NCH, _CH), lambda i: (i, 0, 0)),
        out_shape=jax.ShapeDtypeStruct((_NW, _NCH, _CH), jnp.int32),
    )(ei3)


def _tc_body(p_ref, wt_ref, b_ref, o_ref):
    x = p_ref[0] + p_ref[1]
    y = jnp.dot(x, wt_ref[...], preferred_element_type=jnp.float32)
    o_ref[...] = jnp.maximum(y + b_ref[...], 0.0)


def _tc_linear(partials, Wt, b2):
    bm = 1000
    return pl.pallas_call(
        _tc_body,
        grid=(_N_NODES // bm,),
        in_specs=[
            pl.BlockSpec((_NC, bm, _D), lambda i: (0, i, 0)),
            pl.BlockSpec((_D, _D), lambda i: (0, 0)),
            pl.BlockSpec((1, _D), lambda i: (0, 0)),
        ],
        out_specs=pl.BlockSpec((bm, _D), lambda i: (i, 0)),
        out_shape=jax.ShapeDtypeStruct((_N_NODES, _D), jnp.float32),
    )(partials, Wt, b2)


def kernel(feature, edge_index, W, b):
    n_edges = edge_index.shape[1]
    ei3 = edge_index.astype(jnp.int32).reshape(2, n_edges // _CH, _CH)
    pk3 = _pack_edges(ei3, n_edges)
    partials = _sc_aggregate(feature, pk3)
    return _tc_linear(partials, W.T, b.reshape(1, _D))


# speedup vs baseline: 1.0022x; 1.0022x over previous
import functools

import jax
import jax.numpy as jnp
from jax import lax
from jax.experimental import pallas as pl
from jax.experimental.pallas import tpu as pltpu
from jax.experimental.pallas import tpu_sc as plsc

_NC = 2
_NS = 16
_NW = _NC * _NS

_N_NODES = 10000
_D = 128
_CH = 128
_NCH = 80
_E_PAD = _NW * _NCH * _CH
_ACC_N = 10240
_STRIPE = _ACC_N // _NS
_DUMMY = _N_NODES


def _sc_body(feat, pk, out, pkv, srcb, dstb, rows, acc, gs0, gs1, ss0, ss1):
    c = lax.axis_index("c")
    s = lax.axis_index("s")
    slab = c * _NS + s
    gsems = (gs0, gs1)
    ssems = (ss0, ss1)

    zeros16 = jnp.zeros((16,), jnp.float32)

    def zrow(r, carry):
        for g in range(8):
            rows[0, r, pl.ds(g * 16, 16)] = zeros16
        return carry

    lax.fori_loop(0, _CH, zrow, 0)
    base = s * _STRIPE
    for i in range(_STRIPE // _CH):
        pltpu.sync_copy(rows.at[0], acc.at[pl.ds(base + i * _CH, _CH)])
    plsc.subcore_barrier()

    pltpu.sync_copy(pk.at[slab], pkv)

    def unpack(j, b):
        for g in range(8):
            v = pkv[j, pl.ds(g * 16, 16)]
            srcb[b, pl.ds(g * 16, 16)] = v & 0xFFFF
            dstb[b, pl.ds(g * 16, 16)] = v >> 16

    def gather_start(b):
        pltpu.async_copy(feat.at[srcb.at[b]], rows.at[b], gsems[b])

    def gather_wait(b):
        pltpu.make_async_copy(feat.at[srcb.at[b]], rows.at[b], gsems[b]).wait()

    def scatter_start(b):
        pltpu.async_copy(rows.at[b], acc.at[dstb.at[b]], ssems[b], add=True)

    def scatter_wait(b):
        pltpu.make_async_copy(rows.at[b], acc.at[dstb.at[b]], ssems[b]).wait()

    for b in range(2):
        unpack(b, b)
        gather_start(b)

    def body(i, carry):
        for b in range(2):
            j = i * 2 + b
            gather_wait(b)
            scatter_start(b)
            nxt = j + 2

            @pl.when(nxt < _NCH)
            def _():
                scatter_wait(b)
                unpack(nxt, b)
                gather_start(b)

        return carry

    lax.fori_loop(0, _NCH // 2, body, 0)
    for b in range(2):
        scatter_wait(b)

    plsc.subcore_barrier()
    pltpu.sync_copy(acc.at[pl.ds(base, _STRIPE)], out.at[c, pl.ds(base, _STRIPE)])


@jax.jit
def _sc_aggregate(feature, pk3):
    mesh = plsc.VectorSubcoreMesh(core_axis_name="c", subcore_axis_name="s")
    k = functools.partial(
        pl.kernel,
        mesh=mesh,
        out_type=jax.ShapeDtypeStruct((_NC, _ACC_N, _D), jnp.float32),
        scratch_types=[
            pltpu.VMEM((_NCH, _CH), jnp.int32),
            pltpu.VMEM((2, _CH), jnp.int32),
            pltpu.VMEM((2, _CH), jnp.int32),
            pltpu.VMEM((2, _CH, _D), jnp.float32),
            pltpu.VMEM_SHARED((_ACC_N, _D), jnp.float32),
            pltpu.SemaphoreType.DMA,
            pltpu.SemaphoreType.DMA,
            pltpu.SemaphoreType.DMA,
            pltpu.SemaphoreType.DMA,
        ],
    )(_sc_body)
    return k(feature, pk3)


def _pack_edges(ei3, n_edges):
    def body(e_ref, o_ref):
        e = e_ref[...]
        src = e[0]
        dst = e[1]
        pid = pl.program_id(0)
        first_pad_block = n_edges // (_NCH * _CH)

        @pl.when(pid < first_pad_block)
        def _():
            o_ref[0] = src | (dst << 16)

        @pl.when(pid >= first_pad_block)
        def _():
            rows_i = jax.lax.broadcasted_iota(jnp.int32, (_NCH, _CH), 0)
            cols_i = jax.lax.broadcasted_iota(jnp.int32, (_NCH, _CH), 1)
            gid = pid * (_NCH * _CH) + rows_i * _CH + cols_i
            valid = gid < n_edges
            s2 = jnp.where(valid, src, gid % _N_NODES)
            d2 = jnp.where(valid, dst, _DUMMY + gid % (_ACC_N - _N_NODES))
            o_ref[0] = s2 | (d2 << 16)

    return pl.pallas_call(
        body,
        grid=(_NW,),
        in_specs=[pl.BlockSpec((2, _NCH, _CH), lambda i: (0, i, 0))],
        out_specs=pl.BlockSpec((1, _NCH, _CH), lambda i: (i, 0, 0)),
        out_shape=jax.ShapeDtypeStruct((_NW, _NCH, _CH), jnp.int32),
    )(ei3)


def _tc_body(p_ref, wt_ref, b_ref, o_ref):
    x = p_ref[0] + p_ref[1]
    y = jnp.dot(x, wt_ref[...], preferred_element_type=jnp.float32)
    o_ref[...] = jnp.maximum(y + b_ref[...], 0.0)


def _tc_linear(partials, Wt, b2):
    bm = 1000
    return pl.pallas_call(
        _tc_body,
        grid=(_N_NODES // bm,),
        in_specs=[
            pl.BlockSpec((_NC, bm, _D), lambda i: (0, i, 0)),
            pl.BlockSpec((_D, _D), lambda i: (0, 0)),
            pl.BlockSpec((1, _D), lambda i: (0, 0)),
        ],
        out_specs=pl.BlockSpec((bm, _D), lambda i: (i, 0)),
        out_shape=jax.ShapeDtypeStruct((_N_NODES, _D), jnp.float32),
    )(partials, Wt, b2)


def kernel(feature, edge_index, W, b):
    n_edges = edge_index.shape[1]
    ei3 = edge_index.astype(jnp.int32).reshape(2, n_edges // _CH, _CH)
    pk3 = _pack_edges(ei3, n_edges)
    partials = _sc_aggregate(feature, pk3)
    return _tc_linear(partials, W.T, b.reshape(1, _D))


# speedup vs baseline: 1.0962x; 1.0938x over previous
import functools

import jax
import jax.numpy as jnp
from jax import lax
from jax.experimental import pallas as pl
from jax.experimental.pallas import tpu as pltpu
from jax.experimental.pallas import tpu_sc as plsc

_NC = 2
_NS = 16
_NW = _NC * _NS

_N_NODES = 10000
_D = 128
_CH = 128
_NCH = 80
_E_PAD = _NW * _NCH * _CH
_ACC_N = 10240
_STRIPE = _ACC_N // _NS
_DUMMY = _N_NODES


def _sc_body(feat, pk, out, pkv, srcb, dstb, rows, acc, gs0, gs1, ss0, ss1):
    c = lax.axis_index("c")
    s = lax.axis_index("s")
    slab = c * _NS + s
    gsems = (gs0, gs1)
    ssems = (ss0, ss1)

    zeros16 = jnp.zeros((16,), jnp.float32)

    def zrow(r, carry):
        for g in range(8):
            rows[0, r, pl.ds(g * 16, 16)] = zeros16
        return carry

    lax.fori_loop(0, _CH, zrow, 0)
    base = s * _STRIPE
    for i in range(_STRIPE // _CH):
        pltpu.sync_copy(rows.at[0], acc.at[pl.ds(base + i * _CH, _CH)])
    plsc.subcore_barrier()

    pltpu.sync_copy(pk.at[slab], pkv)

    def unpack(j, b):
        for g in range(8):
            v = pkv[j, pl.ds(g * 16, 16)]
            srcb[b, pl.ds(g * 16, 16)] = v & 0xFFFF
            dstb[b, pl.ds(g * 16, 16)] = v >> 16

    def gather_start(b):
        pltpu.async_copy(feat.at[srcb.at[b]], rows.at[b], gsems[b])

    def gather_wait(b):
        pltpu.make_async_copy(feat.at[srcb.at[b]], rows.at[b], gsems[b]).wait()

    def scatter_start(b):
        pltpu.async_copy(rows.at[b], acc.at[dstb.at[b]], ssems[b], add=True)

    def scatter_wait(b):
        pltpu.make_async_copy(rows.at[b], acc.at[dstb.at[b]], ssems[b]).wait()

    for b in range(2):
        unpack(b, b)
        gather_start(b)

    def body(i, carry):
        for b in range(2):
            j = i * 2 + b
            gather_wait(b)
            scatter_start(b)
            nxt = j + 2

            @pl.when(nxt < _NCH)
            def _():
                scatter_wait(b)
                unpack(nxt, b)
                gather_start(b)

        return carry

    lax.fori_loop(0, _NCH // 2, body, 0)
    for b in range(2):
        scatter_wait(b)

    plsc.subcore_barrier()
    pltpu.sync_copy(acc.at[pl.ds(base, _STRIPE)], out.at[c, pl.ds(base, _STRIPE)])


@jax.jit
def _sc_aggregate(feature, pk3):
    mesh = plsc.VectorSubcoreMesh(core_axis_name="c", subcore_axis_name="s")
    k = functools.partial(
        pl.kernel,
        mesh=mesh,
        out_type=jax.ShapeDtypeStruct((_NC, _ACC_N, _D), jnp.float32),
        scratch_types=[
            pltpu.VMEM((_NCH, _CH), jnp.int32),
            pltpu.VMEM((2, _CH), jnp.int32),
            pltpu.VMEM((2, _CH), jnp.int32),
            pltpu.VMEM((2, _CH, _D), jnp.float32),
            pltpu.VMEM_SHARED((_ACC_N, _D), jnp.float32),
            pltpu.SemaphoreType.DMA,
            pltpu.SemaphoreType.DMA,
            pltpu.SemaphoreType.DMA,
            pltpu.SemaphoreType.DMA,
        ],
    )(_sc_body)
    return k(feature, pk3)


def _pack_edges(ei3, n_edges):
    def body(e_ref, o_ref):
        e = e_ref[...]
        src = e[0]
        dst = e[1]
        pid = pl.program_id(0)
        first_pad_block = n_edges // (8 * _NCH * _CH)

        @pl.when(pid < first_pad_block)
        def _():
            o_ref[...] = (src | (dst << 16)).reshape(8, _NCH, _CH)

        @pl.when(pid >= first_pad_block)
        def _():
            rows_i = jax.lax.broadcasted_iota(jnp.int32, (8 * _NCH, _CH), 0)
            cols_i = jax.lax.broadcasted_iota(jnp.int32, (8 * _NCH, _CH), 1)
            gid = pid * (8 * _NCH * _CH) + rows_i * _CH + cols_i
            valid = gid < n_edges
            s2 = jnp.where(valid, src, gid % _N_NODES)
            d2 = jnp.where(valid, dst, _DUMMY + gid % (_ACC_N - _N_NODES))
            o_ref[...] = (s2 | (d2 << 16)).reshape(8, _NCH, _CH)

    return pl.pallas_call(
        body,
        grid=(_NW // 8,),
        in_specs=[pl.BlockSpec((2, 8 * _NCH, _CH), lambda i: (0, i, 0))],
        out_specs=pl.BlockSpec((8, _NCH, _CH), lambda i: (i, 0, 0)),
        out_shape=jax.ShapeDtypeStruct((_NW, _NCH, _CH), jnp.int32),
    )(ei3)


def _tc_body(p_ref, wt_ref, b_ref, o_ref):
    x = p_ref[0] + p_ref[1]
    y = jnp.dot(x, wt_ref[...], preferred_element_type=jnp.float32)
    o_ref[...] = jnp.maximum(y + b_ref[...], 0.0)


def _tc_linear(partials, Wt, b2):
    bm = 1000
    return pl.pallas_call(
        _tc_body,
        grid=(_N_NODES // bm,),
        in_specs=[
            pl.BlockSpec((_NC, bm, _D), lambda i: (0, i, 0)),
            pl.BlockSpec((_D, _D), lambda i: (0, 0)),
            pl.BlockSpec((1, _D), lambda i: (0, 0)),
        ],
        out_specs=pl.BlockSpec((bm, _D), lambda i: (i, 0)),
        out_shape=jax.ShapeDtypeStruct((_N_NODES, _D), jnp.float32),
    )(partials, Wt, b2)


def kernel(feature, edge_index, W, b):
    n_edges = edge_index.shape[1]
    ei3 = edge_index.astype(jnp.int32).reshape(2, n_edges // _CH, _CH)
    pk3 = _pack_edges(ei3, n_edges)
    partials = _sc_aggregate(feature, pk3)
    return _tc_linear(partials, W.T, b.reshape(1, _D))


# speedup vs baseline: 1.1223x; 1.0238x over previous
import functools

import jax
import jax.numpy as jnp
from jax import lax
from jax.experimental import pallas as pl
from jax.experimental.pallas import tpu as pltpu
from jax.experimental.pallas import tpu_sc as plsc

_NC = 2
_NS = 16
_NW = _NC * _NS

_N_NODES = 10000
_D = 128
_CH = 128
_NCH = 80
_E_PAD = _NW * _NCH * _CH
_ACC_N = 10240
_STRIPE = _ACC_N // _NS
_DUMMY = _N_NODES


def _sc_body(feat, pk, out, pkv, srcb, dstb, rows, acc, gs0, gs1, ss0, ss1):
    c = lax.axis_index("c")
    s = lax.axis_index("s")
    slab = c * _NS + s
    gsems = (gs0, gs1)
    ssems = (ss0, ss1)

    zeros16 = jnp.zeros((16,), jnp.float32)

    def zrow(r, carry):
        for g in range(8):
            rows[0, r, pl.ds(g * 16, 16)] = zeros16
        return carry

    lax.fori_loop(0, _CH, zrow, 0)
    base = s * _STRIPE
    for i in range(_STRIPE // _CH):
        pltpu.sync_copy(rows.at[0], acc.at[pl.ds(base + i * _CH, _CH)])
    plsc.subcore_barrier()

    pltpu.sync_copy(pk.at[slab], pkv)

    def unpack(j, b):
        for g in range(8):
            v = pkv[j, pl.ds(g * 16, 16)]
            srcb[b, pl.ds(g * 16, 16)] = v & 0xFFFF
            dstb[b, pl.ds(g * 16, 16)] = v >> 16

    def gather_start(b):
        pltpu.async_copy(feat.at[srcb.at[b]], rows.at[b], gsems[b])

    def gather_wait(b):
        pltpu.make_async_copy(feat.at[srcb.at[b]], rows.at[b], gsems[b]).wait()

    def scatter_start(b):
        pltpu.async_copy(rows.at[b], acc.at[dstb.at[b]], ssems[b], add=True)

    def scatter_wait(b):
        pltpu.make_async_copy(rows.at[b], acc.at[dstb.at[b]], ssems[b]).wait()

    for b in range(2):
        unpack(b, b)
        gather_start(b)

    def body(i, carry):
        for b in range(2):
            j = i * 2 + b
            gather_wait(b)
            scatter_start(b)
            nxt = j + 2

            @pl.when(nxt < _NCH)
            def _():
                scatter_wait(b)
                unpack(nxt, b)
                gather_start(b)

        return carry

    lax.fori_loop(0, _NCH // 2, body, 0)
    for b in range(2):
        scatter_wait(b)

    plsc.subcore_barrier()
    pltpu.sync_copy(acc.at[pl.ds(base, _STRIPE)], out.at[c, pl.ds(base, _STRIPE)])


@jax.jit
def _sc_aggregate(feature, pk3):
    mesh = plsc.VectorSubcoreMesh(core_axis_name="c", subcore_axis_name="s")
    k = functools.partial(
        pl.kernel,
        mesh=mesh,
        out_type=jax.ShapeDtypeStruct((_NC, _ACC_N, _D), jnp.float32),
        scratch_types=[
            pltpu.VMEM((_NCH, _CH), jnp.int32),
            pltpu.VMEM((2, _CH), jnp.int32),
            pltpu.VMEM((2, _CH), jnp.int32),
            pltpu.VMEM((2, _CH, _D), jnp.float32),
            pltpu.VMEM_SHARED((_ACC_N, _D), jnp.float32),
            pltpu.SemaphoreType.DMA,
            pltpu.SemaphoreType.DMA,
            pltpu.SemaphoreType.DMA,
            pltpu.SemaphoreType.DMA,
        ],
    )(_sc_body)
    return k(feature, pk3)


def _pack_edges(ei3, n_edges):
    def body(e_ref, o_ref):
        e = e_ref[...]
        src = e[0]
        dst = e[1]
        pid = pl.program_id(0)
        first_pad_block = n_edges // (16 * _NCH * _CH)

        @pl.when(pid < first_pad_block)
        def _():
            o_ref[...] = (src | (dst << 16)).reshape(16, _NCH, _CH)

        @pl.when(pid >= first_pad_block)
        def _():
            rows_i = jax.lax.broadcasted_iota(jnp.int32, (16 * _NCH, _CH), 0)
            cols_i = jax.lax.broadcasted_iota(jnp.int32, (16 * _NCH, _CH), 1)
            gid = pid * (16 * _NCH * _CH) + rows_i * _CH + cols_i
            valid = gid < n_edges
            s2 = jnp.where(valid, src, gid % _N_NODES)
            d2 = jnp.where(valid, dst, _DUMMY + gid % (_ACC_N - _N_NODES))
            o_ref[...] = (s2 | (d2 << 16)).reshape(16, _NCH, _CH)

    return pl.pallas_call(
        body,
        grid=(_NW // 16,),
        in_specs=[pl.BlockSpec((2, 16 * _NCH, _CH), lambda i: (0, i, 0))],
        out_specs=pl.BlockSpec((16, _NCH, _CH), lambda i: (i, 0, 0)),
        out_shape=jax.ShapeDtypeStruct((_NW, _NCH, _CH), jnp.int32),
    )(ei3)


def _tc_body(p_ref, wt_ref, b_ref, o_ref):
    x = p_ref[0] + p_ref[1]
    y = jnp.dot(x, wt_ref[...], preferred_element_type=jnp.float32)
    o_ref[...] = jnp.maximum(y + b_ref[...], 0.0)


def _tc_linear(partials, Wt, b2):
    bm = 2000
    return pl.pallas_call(
        _tc_body,
        grid=(_N_NODES // bm,),
        in_specs=[
            pl.BlockSpec((_NC, bm, _D), lambda i: (0, i, 0)),
            pl.BlockSpec((_D, _D), lambda i: (0, 0)),
            pl.BlockSpec((1, _D), lambda i: (0, 0)),
        ],
        out_specs=pl.BlockSpec((bm, _D), lambda i: (i, 0)),
        out_shape=jax.ShapeDtypeStruct((_N_NODES, _D), jnp.float32),
    )(partials, Wt, b2)


def kernel(feature, edge_index, W, b):
    n_edges = edge_index.shape[1]
    ei3 = edge_index.astype(jnp.int32).reshape(2, n_edges // _CH, _CH)
    pk3 = _pack_edges(ei3, n_edges)
    partials = _sc_aggregate(feature, pk3)
    return _tc_linear(partials, W.T, b.reshape(1, _D))
